# baseline (speedup 1.0000x reference)
import jax
import jax.numpy as jnp
from jax.experimental import pallas as pl
from jax.experimental.pallas import tpu as pltpu

N = 10000
DEG = 32
D = 128
BLOCK = 400
DSPLIT = 2
DCH = DEG // DSPLIT


def _fused_body(src_ref, neigh_ref, w_ref, b_ref, out_ref, acc_ref):
    j = pl.program_id(1)
    part = jnp.sum(neigh_ref[...], axis=1)

    @pl.when(j == 0)
    def _():
        acc_ref[...] = part

    @pl.when(j > 0)
    def _():
        acc_ref[...] += part

    @pl.when(j == DSPLIT - 1)
    def _():
        agg = acc_ref[...] * (1.0 / DEG)
        h = jnp.dot(agg, w_ref[...], preferred_element_type=jnp.float32)
        h += jnp.dot(src_ref[...], b_ref[...],
                     preferred_element_type=jnp.float32)
        out_ref[...] = jnp.maximum(h, 0.0)


def kernel(src_node_features, neighbor_node_features, W_agg, b):
    return pl.pallas_call(
        _fused_body,
        grid=(N // BLOCK, DSPLIT),
        in_specs=[
            pl.BlockSpec((BLOCK, D), lambda i, j: (i, 0)),
            pl.BlockSpec((BLOCK, DCH, D), lambda i, j: (i, j, 0)),
            pl.BlockSpec((D, D), lambda i, j: (0, 0)),
            pl.BlockSpec((D, D), lambda i, j: (0, 0)),
        ],
        out_specs=pl.BlockSpec((BLOCK, D), lambda i, j: (i, 0)),
        out_shape=jax.ShapeDtypeStruct((N, D), jnp.float32),
        scratch_shapes=[pltpu.VMEM((BLOCK, D), jnp.float32)],
    )(src_node_features, neighbor_node_features, W_agg, b)


# speedup vs baseline: 1.3591x; 1.3591x over previous
import jax
import jax.numpy as jnp
from jax.experimental import pallas as pl

N = 10000
DEG = 32
D = 128
BLOCK = 400


def _fused_body(src_ref, neigh_ref, w_ref, b_ref, out_ref):
    agg = jnp.sum(neigh_ref[...], axis=1) * (1.0 / DEG)
    h = jnp.dot(agg, w_ref[...], preferred_element_type=jnp.float32)
    h += jnp.dot(src_ref[...], b_ref[...], preferred_element_type=jnp.float32)
    out_ref[...] = jnp.maximum(h, 0.0)


def kernel(src_node_features, neighbor_node_features, W_agg, b):
    grid = N // BLOCK
    return pl.pallas_call(
        _fused_body,
        grid=(grid,),
        in_specs=[
            pl.BlockSpec((BLOCK, D), lambda i: (i, 0)),
            pl.BlockSpec((BLOCK, DEG, D), lambda i: (i, 0, 0)),
            pl.BlockSpec((D, D), lambda i: (0, 0)),
            pl.BlockSpec((D, D), lambda i: (0, 0)),
        ],
        out_specs=pl.BlockSpec((BLOCK, D), lambda i: (i, 0)),
        out_shape=jax.ShapeDtypeStruct((N, D), jnp.float32),
    )(src_node_features, neighbor_node_features, W_agg, b)
